# baseline (speedup 1.0000x reference)
import functools

import jax
import jax.numpy as jnp
from jax import lax
from jax.experimental import pallas as pl
from jax.experimental.pallas import tpu as pltpu
from jax.experimental.pallas import tpu_sc as plsc

N = 10000
E = 160000
G = 16
F = 512
FC = 128
NPASS = F // FC
EC = 80
EPAD = EC * 2048 - E
N_ACC = 10240
STRIPE = N_ACC // 16
BI = 1000
NBLK = N // BI

_mesh = plsc.VectorSubcoreMesh(
    core_axis_name="c", subcore_axis_name="s", num_cores=2, num_subcores=16)


def _zero_vmem(ref, nrows, ncols):
    zv = jnp.zeros((16,), jnp.float32)

    def body(i, _):
        for j in range(ncols // 16):
            ref[i, pl.ds(j * 16, 16)] = zv
        return 0

    lax.fori_loop(0, nrows, body, 0)


@functools.partial(
    pl.kernel,
    out_type=jax.ShapeDtypeStruct((2, N_ACC, FC), jnp.float32),
    mesh=_mesh,
    scratch_types=[
        pltpu.VMEM((8, 128), jnp.int32),
        pltpu.VMEM((128, FC), jnp.float32),
        pltpu.VMEM((128, FC), jnp.float32),
        pltpu.VMEM_SHARED((N_ACC, FC), jnp.float32),
    ],
)
def _sc_degree(dst2h, deg_out, didx_v, ones_v, stage_v, acc_sh):
    c = lax.axis_index("c")
    s = lax.axis_index("s")

    _zero_vmem(stage_v, 128, FC)
    ov = jnp.ones((16,), jnp.float32)

    def fill_ones(i, _):
        for j in range(FC // 16):
            ones_v[i, pl.ds(j * 16, 16)] = ov
        return 0

    lax.fori_loop(0, 128, fill_ones, 0)

    for t in range(STRIPE // 128):
        pltpu.sync_copy(stage_v, acc_sh.at[pl.ds(s * STRIPE + t * 128, 128)])
    plsc.subcore_barrier()

    w = s * 2 + c

    def step(k, _):
        j = w + 32 * k
        pltpu.sync_copy(dst2h.at[j], didx_v)
        for r in range(8):
            pltpu.sync_copy(ones_v, acc_sh.at[didx_v.at[r]], add=True)
        return 0

    lax.fori_loop(0, EC * 2 // 32, step, 0)
    plsc.subcore_barrier()

    for t in range(STRIPE // 128):
        r0 = s * STRIPE + t * 128
        pltpu.sync_copy(acc_sh.at[pl.ds(r0, 128)], stage_v)
        pltpu.sync_copy(stage_v, deg_out.at[c, pl.ds(r0, 128)])


@functools.partial(
    pl.kernel,
    out_type=jax.ShapeDtypeStruct((NPASS, N_ACC, FC), jnp.float32),
    mesh=_mesh,
    scratch_types=[
        pltpu.VMEM((16, 128), jnp.int32),
        pltpu.VMEM((16, 128), jnp.int32),
        pltpu.VMEM((2, 128, FC), jnp.float32),
        pltpu.VMEM((64, FC), jnp.float32),
        pltpu.VMEM_SHARED((N_ACC, FC), jnp.float32),
        pltpu.SemaphoreType.DMA,
    ],
)
def _sc_agg(h4, src2, dst2, agg4, sidx_v, didx_v, rows_v, stage_v,
            acc_sh, sem):
    c = lax.axis_index("c")
    s = lax.axis_index("s")
    _zero_vmem(stage_v, 64, FC)

    def run_pass(p):
        for t in range(STRIPE // 64):
            pltpu.sync_copy(stage_v,
                            acc_sh.at[pl.ds(s * STRIPE + t * 64, 64)])
        plsc.subcore_barrier()

        def step(k, _):
            j = s + 16 * k
            pltpu.sync_copy(src2.at[j], sidx_v)
            pltpu.sync_copy(dst2.at[j], didx_v)
            pltpu.async_copy(h4.at[p].at[sidx_v.at[0]], rows_v.at[0], sem)
            for r in range(16):
                if r < 15:
                    pltpu.async_copy(h4.at[p].at[sidx_v.at[r + 1]],
                                     rows_v.at[(r + 1) % 2], sem)
                pltpu.make_async_copy(h4.at[p].at[sidx_v.at[r]],
                                      rows_v.at[r % 2], sem).wait()
                pltpu.sync_copy(rows_v.at[r % 2], acc_sh.at[didx_v.at[r]],
                                add=True)
            return 0

        lax.fori_loop(0, EC // 16, step, 0)
        plsc.subcore_barrier()

        for t in range(STRIPE // 64):
            r0 = s * STRIPE + t * 64
            pltpu.sync_copy(acc_sh.at[pl.ds(r0, 64)], stage_v)
            pltpu.sync_copy(stage_v, agg4.at[p, pl.ds(r0, 64)])
        plsc.subcore_barrier()
        _zero_vmem(stage_v, 64, FC)
        plsc.subcore_barrier()

    for p_local in range(2):
        for cval in range(2):
            @pl.when(c == cval)
            def _(p=2 * cval + p_local):
                run_pass(p)


def _dis_block(deg_ref):
    deg = deg_ref[0, :, 0:1] + deg_ref[1, :, 0:1] + 1.0
    return lax.rsqrt(deg)


def _embed_body(nx_ref, emb_ref, waa_ref, baa_ref, out_ref):
    nx = nx_ref[...]
    oh = (nx == lax.broadcasted_iota(jnp.int32, (BI, 32), 1)
          ).astype(jnp.float32)
    table = jnp.dot(emb_ref[...], waa_ref[...],
                    preferred_element_type=jnp.float32)
    x0 = jnp.dot(oh, table, preferred_element_type=jnp.float32) + baa_ref[...]
    out_ref[...] = jnp.maximum(x0, 0.0)


def _embed(nx, emb32, waa, baa):
    return pl.pallas_call(
        _embed_body,
        grid=(NBLK,),
        in_specs=[
            pl.BlockSpec((BI, 1), lambda i: (i, 0)),
            pl.BlockSpec((32, 96), lambda i: (0, 0)),
            pl.BlockSpec((96, F), lambda i: (0, 0)),
            pl.BlockSpec((1, F), lambda i: (0, 0)),
        ],
        out_specs=pl.BlockSpec((BI, F), lambda i: (i, 0)),
        out_shape=jax.ShapeDtypeStruct((N, F), jnp.float32),
    )(nx, emb32, waa, baa)


def _matmul_body(x_ref, w_ref, deg_ref, h4_ref):
    dis = _dis_block(deg_ref)
    h = jnp.dot(x_ref[...], w_ref[...], preferred_element_type=jnp.float32)
    hp = h * dis
    for p in range(NPASS):
        h4_ref[p] = hp[:, p * FC:(p + 1) * FC]


def _matmul_scale(x, w, deg_part):
    return pl.pallas_call(
        _matmul_body,
        grid=(NBLK,),
        in_specs=[
            pl.BlockSpec((BI, F), lambda i: (i, 0)),
            pl.BlockSpec((F, F), lambda i: (0, 0)),
            pl.BlockSpec((2, BI, FC), lambda i: (0, i, 0)),
        ],
        out_specs=pl.BlockSpec((NPASS, BI, FC), lambda i: (0, i, 0)),
        out_shape=jax.ShapeDtypeStruct((NPASS, N, FC), jnp.float32),
    )(x, w, deg_part)


def _epilogue_body(agg_ref, h4_ref, deg_ref, b_ref, res_ref, out_ref,
                   *, has_res):
    dis = _dis_block(deg_ref)
    agg = jnp.concatenate([agg_ref[p] for p in range(NPASS)], axis=1)
    hp = jnp.concatenate([h4_ref[p] for p in range(NPASS)], axis=1)
    y = jnp.maximum(dis * (agg + hp) + b_ref[...], 0.0)
    if has_res:
        y = y + res_ref[...]
    out_ref[...] = y


def _epilogue(agg4, h4, deg_part, b, res, has_res):
    body = functools.partial(_epilogue_body, has_res=has_res)
    return pl.pallas_call(
        body,
        grid=(NBLK,),
        in_specs=[
            pl.BlockSpec((NPASS, BI, FC), lambda i: (0, i, 0)),
            pl.BlockSpec((NPASS, BI, FC), lambda i: (0, i, 0)),
            pl.BlockSpec((2, BI, FC), lambda i: (0, i, 0)),
            pl.BlockSpec((1, F), lambda i: (0, 0)),
            pl.BlockSpec((BI, F), lambda i: (i, 0)),
        ],
        out_specs=pl.BlockSpec((BI, F), lambda i: (i, 0)),
        out_shape=jax.ShapeDtypeStruct((N, F), jnp.float32),
    )(agg4, h4, deg_part, b, res)


def _readout_body(x_ref, batch_ref, wr1_ref, br1_ref, wr2_ref, br2_ref,
                  y_ref, g_ref):
    i = pl.program_id(0)

    @pl.when(i == 0)
    def _():
        g_ref[...] = jnp.full((G, F), -jnp.inf, jnp.float32)

    bb = batch_ref[...]
    xb = x_ref[...]
    for g in range(G):
        m = bb == g
        mg = jnp.max(jnp.where(m, xb, -jnp.inf), axis=0, keepdims=True)
        g_ref[pl.ds(g, 1), :] = jnp.maximum(g_ref[pl.ds(g, 1), :], mg)

    @pl.when(i == NBLK - 1)
    def _():
        gg = g_ref[...]
        h = jnp.maximum(
            jnp.dot(gg, wr1_ref[...], preferred_element_type=jnp.float32)
            + br1_ref[...], 0.0)
        y_ref[...] = jax.nn.sigmoid(
            jnp.dot(h, wr2_ref[...], preferred_element_type=jnp.float32)
            + br2_ref[...])


def _readout(x, batch, wr1, br1, wr2, br2):
    return pl.pallas_call(
        _readout_body,
        grid=(NBLK,),
        in_specs=[
            pl.BlockSpec((BI, F), lambda i: (i, 0)),
            pl.BlockSpec((BI, 1), lambda i: (i, 0)),
            pl.BlockSpec((F, 1024), lambda i: (0, 0)),
            pl.BlockSpec((1, 1024), lambda i: (0, 0)),
            pl.BlockSpec((1024, 256), lambda i: (0, 0)),
            pl.BlockSpec((1, 256), lambda i: (0, 0)),
        ],
        out_specs=[
            pl.BlockSpec((G, 256), lambda i: (0, 0)),
            pl.BlockSpec((G, F), lambda i: (0, 0)),
        ],
        out_shape=[
            jax.ShapeDtypeStruct((G, 256), jnp.float32),
            jax.ShapeDtypeStruct((G, F), jnp.float32),
        ],
    )(x, batch, wr1, br1, wr2, br2)


def kernel(native_x, edge_index, batch, emb, W_aa, b_aa, W_g1, b_g1,
           W_g2, b_g2, W_g3, b_g3, W_r1, b_r1, W_r2, b_r2):
    nx = native_x.astype(jnp.int32)
    ei = edge_index.astype(jnp.int32)
    src = jnp.concatenate([ei[0], jnp.zeros((EPAD,), jnp.int32)])
    dst = jnp.concatenate([ei[1], jnp.full((EPAD,), N, jnp.int32)])
    src2 = src.reshape(EC, 16, 128)
    dst2 = dst.reshape(EC, 16, 128)
    emb32 = jnp.pad(emb, ((0, 32 - emb.shape[0]), (0, 0)))

    deg_part = _sc_degree(dst2.reshape(EC * 2, 8, 128))
    x = _embed(nx.reshape(N, 1), emb32, W_aa, b_aa.reshape(1, F))
    for W, b, has_res in ((W_g1, b_g1, False), (W_g2, b_g2, True),
                          (W_g3, b_g3, True)):
        h4 = _matmul_scale(x, W, deg_part)
        agg4 = _sc_agg(h4, src2, dst2)
        x = _epilogue(agg4, h4, deg_part, b.reshape(1, F), x, has_res)
    y_pred, g_level_feat = _readout(x, batch.astype(jnp.int32).reshape(N, 1), W_r1,
                                    b_r1.reshape(1, 1024), W_r2,
                                    b_r2.reshape(1, 256))
    return (y_pred, g_level_feat)


# speedup vs baseline: 1.2271x; 1.2271x over previous
import functools

import jax
import jax.numpy as jnp
from jax import lax
from jax.experimental import pallas as pl
from jax.experimental.pallas import tpu as pltpu
from jax.experimental.pallas import tpu_sc as plsc

N = 10000
E = 160000
G = 16
F = 512
FC = 128
NPASS = F // FC
EC = 160
EPAD = EC * 1024 - E
N_ACC = 10240
STRIPE = N_ACC // 16
BI = 1000
NBLK = N // BI

_mesh = plsc.VectorSubcoreMesh(
    core_axis_name="c", subcore_axis_name="s", num_cores=2, num_subcores=16)


def _zero_vmem(ref, nrows, ncols):
    zv = jnp.zeros((16,), jnp.float32)

    def body(i, _):
        for j in range(ncols // 16):
            ref[i, pl.ds(j * 16, 16)] = zv
        return 0

    lax.fori_loop(0, nrows, body, 0)


@functools.partial(
    pl.kernel,
    out_type=jax.ShapeDtypeStruct((2, N_ACC, FC), jnp.float32),
    mesh=_mesh,
    scratch_types=[
        pltpu.VMEM((8, 128), jnp.int32),
        pltpu.VMEM((128, FC), jnp.float32),
        pltpu.VMEM((128, FC), jnp.float32),
        pltpu.VMEM_SHARED((N_ACC, FC), jnp.float32),
    ],
)
def _sc_degree(dst2h, deg_out, didx_v, ones_v, stage_v, acc_sh):
    c = lax.axis_index("c")
    s = lax.axis_index("s")

    _zero_vmem(stage_v, 128, FC)
    ov = jnp.ones((16,), jnp.float32)

    def fill_ones(i, _):
        for j in range(FC // 16):
            ones_v[i, pl.ds(j * 16, 16)] = ov
        return 0

    lax.fori_loop(0, 128, fill_ones, 0)

    for t in range(STRIPE // 128):
        pltpu.sync_copy(stage_v, acc_sh.at[pl.ds(s * STRIPE + t * 128, 128)])
    plsc.subcore_barrier()

    w = s * 2 + c

    def step(k, _):
        j = w + 32 * k
        pltpu.sync_copy(dst2h.at[j], didx_v)
        for r in range(8):
            pltpu.sync_copy(ones_v, acc_sh.at[didx_v.at[r]], add=True)
        return 0

    lax.fori_loop(0, EC // 32, step, 0)
    plsc.subcore_barrier()

    for t in range(STRIPE // 128):
        r0 = s * STRIPE + t * 128
        pltpu.sync_copy(acc_sh.at[pl.ds(r0, 128)], stage_v)
        pltpu.sync_copy(stage_v, deg_out.at[c, pl.ds(r0, 128)])


@functools.partial(
    pl.kernel,
    out_type=jax.ShapeDtypeStruct((NPASS, N_ACC, FC), jnp.float32),
    mesh=_mesh,
    scratch_types=[
        pltpu.VMEM((8, 128), jnp.int32),
        pltpu.VMEM((8, 128), jnp.int32),
        pltpu.VMEM((2, 128, FC), jnp.float32),
        pltpu.VMEM((64, FC), jnp.float32),
        pltpu.VMEM_SHARED((N_ACC, FC), jnp.float32),
        pltpu.SemaphoreType.DMA,
        pltpu.SemaphoreType.DMA,
    ],
)
def _sc_agg(h4, src2, dst2, agg4, sidx_v, didx_v, rows_v, stage_v,
            acc_sh, sem, sem2):
    c = lax.axis_index("c")
    s = lax.axis_index("s")
    _zero_vmem(stage_v, 64, FC)

    def run_pass(p):
        for t in range(STRIPE // 64):
            pltpu.sync_copy(stage_v,
                            acc_sh.at[pl.ds(s * STRIPE + t * 64, 64)])
        plsc.subcore_barrier()

        def step(k, _):
            j = s + 16 * k
            pltpu.sync_copy(src2.at[j], sidx_v)
            pltpu.sync_copy(dst2.at[j], didx_v)
            pltpu.async_copy(h4.at[p].at[sidx_v.at[0]], rows_v.at[0], sem)
            for r in range(8):
                if r < 7:
                    if r >= 1:
                        pltpu.make_async_copy(
                            rows_v.at[(r + 1) % 2],
                            acc_sh.at[didx_v.at[r - 1]], sem2).wait()
                    pltpu.async_copy(h4.at[p].at[sidx_v.at[r + 1]],
                                     rows_v.at[(r + 1) % 2], sem)
                pltpu.make_async_copy(h4.at[p].at[sidx_v.at[r]],
                                      rows_v.at[r % 2], sem).wait()
                pltpu.async_copy(rows_v.at[r % 2], acc_sh.at[didx_v.at[r]],
                                sem2, add=True)
            pltpu.make_async_copy(rows_v.at[0], acc_sh.at[didx_v.at[6]],
                                  sem2).wait()
            pltpu.make_async_copy(rows_v.at[1], acc_sh.at[didx_v.at[7]],
                                  sem2).wait()
            return 0

        lax.fori_loop(0, EC // 16, step, 0)
        plsc.subcore_barrier()

        for t in range(STRIPE // 64):
            r0 = s * STRIPE + t * 64
            pltpu.sync_copy(acc_sh.at[pl.ds(r0, 64)], stage_v)
            pltpu.sync_copy(stage_v, agg4.at[p, pl.ds(r0, 64)])
        plsc.subcore_barrier()
        _zero_vmem(stage_v, 64, FC)
        plsc.subcore_barrier()

    for p_local in range(2):
        for cval in range(2):
            @pl.when(c == cval)
            def _(p=2 * cval + p_local):
                run_pass(p)


def _dis_block(deg_ref):
    deg = deg_ref[0, :, 0:1] + deg_ref[1, :, 0:1] + 1.0
    return lax.rsqrt(deg)


def _embed_body(nx_ref, emb_ref, waa_ref, baa_ref, out_ref):
    nx = nx_ref[...]
    oh = (nx == lax.broadcasted_iota(jnp.int32, (BI, 32), 1)
          ).astype(jnp.float32)
    table = jnp.dot(emb_ref[...], waa_ref[...],
                    preferred_element_type=jnp.float32)
    x0 = jnp.dot(oh, table, preferred_element_type=jnp.float32) + baa_ref[...]
    out_ref[...] = jnp.maximum(x0, 0.0)


def _embed(nx, emb32, waa, baa):
    return pl.pallas_call(
        _embed_body,
        grid=(NBLK,),
        in_specs=[
            pl.BlockSpec((BI, 1), lambda i: (i, 0)),
            pl.BlockSpec((32, 96), lambda i: (0, 0)),
            pl.BlockSpec((96, F), lambda i: (0, 0)),
            pl.BlockSpec((1, F), lambda i: (0, 0)),
        ],
        out_specs=pl.BlockSpec((BI, F), lambda i: (i, 0)),
        out_shape=jax.ShapeDtypeStruct((N, F), jnp.float32),
    )(nx, emb32, waa, baa)


def _matmul_body(x_ref, w_ref, deg_ref, h4_ref):
    dis = _dis_block(deg_ref)
    h = jnp.dot(x_ref[...], w_ref[...], preferred_element_type=jnp.float32)
    hp = h * dis
    for p in range(NPASS):
        h4_ref[p] = hp[:, p * FC:(p + 1) * FC]


def _matmul_scale(x, w, deg_part):
    return pl.pallas_call(
        _matmul_body,
        grid=(NBLK,),
        in_specs=[
            pl.BlockSpec((BI, F), lambda i: (i, 0)),
            pl.BlockSpec((F, F), lambda i: (0, 0)),
            pl.BlockSpec((2, BI, FC), lambda i: (0, i, 0)),
        ],
        out_specs=pl.BlockSpec((NPASS, BI, FC), lambda i: (0, i, 0)),
        out_shape=jax.ShapeDtypeStruct((NPASS, N, FC), jnp.float32),
    )(x, w, deg_part)


def _epilogue_body(agg_ref, h4_ref, deg_ref, b_ref, res_ref, out_ref,
                   *, has_res):
    dis = _dis_block(deg_ref)
    agg = jnp.concatenate([agg_ref[p] for p in range(NPASS)], axis=1)
    hp = jnp.concatenate([h4_ref[p] for p in range(NPASS)], axis=1)
    y = jnp.maximum(dis * (agg + hp) + b_ref[...], 0.0)
    if has_res:
        y = y + res_ref[...]
    out_ref[...] = y


def _epilogue(agg4, h4, deg_part, b, res, has_res):
    body = functools.partial(_epilogue_body, has_res=has_res)
    return pl.pallas_call(
        body,
        grid=(NBLK,),
        in_specs=[
            pl.BlockSpec((NPASS, BI, FC), lambda i: (0, i, 0)),
            pl.BlockSpec((NPASS, BI, FC), lambda i: (0, i, 0)),
            pl.BlockSpec((2, BI, FC), lambda i: (0, i, 0)),
            pl.BlockSpec((1, F), lambda i: (0, 0)),
            pl.BlockSpec((BI, F), lambda i: (i, 0)),
        ],
        out_specs=pl.BlockSpec((BI, F), lambda i: (i, 0)),
        out_shape=jax.ShapeDtypeStruct((N, F), jnp.float32),
    )(agg4, h4, deg_part, b, res)


def _readout_body(x_ref, batch_ref, wr1_ref, br1_ref, wr2_ref, br2_ref,
                  y_ref, g_ref):
    i = pl.program_id(0)

    @pl.when(i == 0)
    def _():
        g_ref[...] = jnp.full((G, F), -jnp.inf, jnp.float32)

    bb = batch_ref[...]
    xb = x_ref[...]
    for g in range(G):
        m = bb == g
        mg = jnp.max(jnp.where(m, xb, -jnp.inf), axis=0, keepdims=True)
        g_ref[pl.ds(g, 1), :] = jnp.maximum(g_ref[pl.ds(g, 1), :], mg)

    @pl.when(i == NBLK - 1)
    def _():
        gg = g_ref[...]
        h = jnp.maximum(
            jnp.dot(gg, wr1_ref[...], preferred_element_type=jnp.float32)
            + br1_ref[...], 0.0)
        y_ref[...] = jax.nn.sigmoid(
            jnp.dot(h, wr2_ref[...], preferred_element_type=jnp.float32)
            + br2_ref[...])


def _readout(x, batch, wr1, br1, wr2, br2):
    return pl.pallas_call(
        _readout_body,
        grid=(NBLK,),
        in_specs=[
            pl.BlockSpec((BI, F), lambda i: (i, 0)),
            pl.BlockSpec((BI, 1), lambda i: (i, 0)),
            pl.BlockSpec((F, 1024), lambda i: (0, 0)),
            pl.BlockSpec((1, 1024), lambda i: (0, 0)),
            pl.BlockSpec((1024, 256), lambda i: (0, 0)),
            pl.BlockSpec((1, 256), lambda i: (0, 0)),
        ],
        out_specs=[
            pl.BlockSpec((G, 256), lambda i: (0, 0)),
            pl.BlockSpec((G, F), lambda i: (0, 0)),
        ],
        out_shape=[
            jax.ShapeDtypeStruct((G, 256), jnp.float32),
            jax.ShapeDtypeStruct((G, F), jnp.float32),
        ],
    )(x, batch, wr1, br1, wr2, br2)


def kernel(native_x, edge_index, batch, emb, W_aa, b_aa, W_g1, b_g1,
           W_g2, b_g2, W_g3, b_g3, W_r1, b_r1, W_r2, b_r2):
    nx = native_x.astype(jnp.int32)
    ei = edge_index.astype(jnp.int32)
    src = jnp.concatenate([ei[0], jnp.zeros((EPAD,), jnp.int32)])
    dst = jnp.concatenate([ei[1], jnp.full((EPAD,), N, jnp.int32)])
    src2 = src.reshape(EC, 8, 128)
    dst2 = dst.reshape(EC, 8, 128)
    emb32 = jnp.pad(emb, ((0, 32 - emb.shape[0]), (0, 0)))

    deg_part = _sc_degree(dst2)
    x = _embed(nx.reshape(N, 1), emb32, W_aa, b_aa.reshape(1, F))
    for W, b, has_res in ((W_g1, b_g1, False), (W_g2, b_g2, True),
                          (W_g3, b_g3, True)):
        h4 = _matmul_scale(x, W, deg_part)
        agg4 = _sc_agg(h4, src2, dst2)
        x = _epilogue(agg4, h4, deg_part, b.reshape(1, F), x, has_res)
    y_pred, g_level_feat = _readout(x, batch.astype(jnp.int32).reshape(N, 1), W_r1,
                                    b_r1.reshape(1, 1024), W_r2,
                                    b_r2.reshape(1, 256))
    return (y_pred, g_level_feat)


# speedup vs baseline: 1.2621x; 1.0285x over previous
import functools

import jax
import jax.numpy as jnp
from jax import lax
from jax.experimental import pallas as pl
from jax.experimental.pallas import tpu as pltpu
from jax.experimental.pallas import tpu_sc as plsc

N = 10000
E = 160000
G = 16
F = 512
FC = 128
NPASS = F // FC
EC = 160
EPAD = EC * 1024 - E
N_ACC = 10240
STRIPE = N_ACC // 16
BI = 1000
NBLK = N // BI

_mesh = plsc.VectorSubcoreMesh(
    core_axis_name="c", subcore_axis_name="s", num_cores=2, num_subcores=16)


def _zero_vmem(ref, nrows, ncols):
    zv = jnp.zeros((16,), jnp.float32)

    def body(i, _):
        for j in range(ncols // 16):
            ref[i, pl.ds(j * 16, 16)] = zv
        return 0

    lax.fori_loop(0, nrows, body, 0)


@functools.partial(
    pl.kernel,
    out_type=jax.ShapeDtypeStruct((2, N_ACC, FC), jnp.float32),
    mesh=_mesh,
    scratch_types=[
        pltpu.VMEM((8, 128), jnp.int32),
        pltpu.VMEM((128, FC), jnp.float32),
        pltpu.VMEM((128, FC), jnp.float32),
        pltpu.VMEM_SHARED((N_ACC, FC), jnp.float32),
    ],
)
def _sc_degree(dst2h, deg_out, didx_v, ones_v, stage_v, acc_sh):
    c = lax.axis_index("c")
    s = lax.axis_index("s")

    _zero_vmem(stage_v, 128, FC)
    ov = jnp.ones((16,), jnp.float32)

    def fill_ones(i, _):
        for j in range(FC // 16):
            ones_v[i, pl.ds(j * 16, 16)] = ov
        return 0

    lax.fori_loop(0, 128, fill_ones, 0)

    for t in range(STRIPE // 128):
        pltpu.sync_copy(stage_v, acc_sh.at[pl.ds(s * STRIPE + t * 128, 128)])
    plsc.subcore_barrier()

    w = s * 2 + c

    def step(k, _):
        j = w + 32 * k
        pltpu.sync_copy(dst2h.at[j], didx_v)
        for r in range(8):
            pltpu.sync_copy(ones_v, acc_sh.at[didx_v.at[r]], add=True)
        return 0

    lax.fori_loop(0, EC // 32, step, 0)
    plsc.subcore_barrier()

    for t in range(STRIPE // 128):
        r0 = s * STRIPE + t * 128
        pltpu.sync_copy(acc_sh.at[pl.ds(r0, 128)], stage_v)
        pltpu.sync_copy(stage_v, deg_out.at[c, pl.ds(r0, 128)])


@functools.partial(
    pl.kernel,
    out_type=jax.ShapeDtypeStruct((NPASS, N_ACC, FC), jnp.float32),
    mesh=_mesh,
    scratch_types=[
        pltpu.VMEM((2, 8, 128), jnp.int32),
        pltpu.VMEM((2, 8, 128), jnp.int32),
        pltpu.VMEM((2, 128, FC), jnp.float32),
        pltpu.VMEM((64, FC), jnp.float32),
        pltpu.VMEM_SHARED((N_ACC, FC), jnp.float32),
        pltpu.SemaphoreType.DMA,
        pltpu.SemaphoreType.DMA,
    ],
)
def _sc_agg(h4, src2, dst2, agg4, sidx_v, didx_v, rows_v, stage_v,
            acc_sh, sem, sem2):
    c = lax.axis_index("c")
    s = lax.axis_index("s")
    _zero_vmem(stage_v, 64, FC)

    def run_pass(p):
        for t in range(STRIPE // 64):
            pltpu.sync_copy(stage_v,
                            acc_sh.at[pl.ds(s * STRIPE + t * 64, 64)])
        plsc.subcore_barrier()

        pltpu.async_copy(src2.at[s], sidx_v.at[0], sem2)
        pltpu.async_copy(dst2.at[s], didx_v.at[0], sem2)
        pltpu.async_copy(src2.at[s + 16], sidx_v.at[1], sem2)
        pltpu.async_copy(dst2.at[s + 16], didx_v.at[1], sem2)

        def step(k, _):
            kp = lax.rem(k, 2)
            j = s + 16 * k
            pltpu.make_async_copy(src2.at[j], sidx_v.at[kp], sem2).wait()
            pltpu.make_async_copy(dst2.at[j], didx_v.at[kp], sem2).wait()
            pltpu.async_copy(h4.at[p].at[sidx_v.at[kp, 0]], rows_v.at[0],
                             sem)
            for r in range(8):
                if r < 7:
                    pltpu.async_copy(h4.at[p].at[sidx_v.at[kp, r + 1]],
                                     rows_v.at[(r + 1) % 2], sem)
                pltpu.make_async_copy(h4.at[p].at[sidx_v.at[kp, r]],
                                      rows_v.at[r % 2], sem).wait()
                pltpu.sync_copy(rows_v.at[r % 2],
                                acc_sh.at[didx_v.at[kp, r]], add=True)

            @pl.when(k + 2 < EC // 16)
            def _():
                j2 = s + 16 * (k + 2)
                pltpu.async_copy(src2.at[j2], sidx_v.at[kp], sem2)
                pltpu.async_copy(dst2.at[j2], didx_v.at[kp], sem2)
            return 0

        lax.fori_loop(0, EC // 16, step, 0)
        plsc.subcore_barrier()

        for t in range(STRIPE // 64):
            r0 = s * STRIPE + t * 64
            pltpu.sync_copy(acc_sh.at[pl.ds(r0, 64)], stage_v)
            pltpu.sync_copy(stage_v, agg4.at[p, pl.ds(r0, 64)])
        plsc.subcore_barrier()
        _zero_vmem(stage_v, 64, FC)
        plsc.subcore_barrier()

    for p_local in range(2):
        for cval in range(2):
            @pl.when(c == cval)
            def _(p=2 * cval + p_local):
                run_pass(p)


def _dis_block(deg_ref):
    deg = deg_ref[0, :, 0:1] + deg_ref[1, :, 0:1] + 1.0
    return lax.rsqrt(deg)


def _embed_body(nx_ref, emb_ref, waa_ref, baa_ref, out_ref):
    nx = nx_ref[...]
    oh = (nx == lax.broadcasted_iota(jnp.int32, (BI, 32), 1)
          ).astype(jnp.float32)
    table = jnp.dot(emb_ref[...], waa_ref[...],
                    preferred_element_type=jnp.float32)
    x0 = jnp.dot(oh, table, preferred_element_type=jnp.float32) + baa_ref[...]
    out_ref[...] = jnp.maximum(x0, 0.0)


def _embed(nx, emb32, waa, baa):
    return pl.pallas_call(
        _embed_body,
        grid=(NBLK,),
        in_specs=[
            pl.BlockSpec((BI, 1), lambda i: (i, 0)),
            pl.BlockSpec((32, 96), lambda i: (0, 0)),
            pl.BlockSpec((96, F), lambda i: (0, 0)),
            pl.BlockSpec((1, F), lambda i: (0, 0)),
        ],
        out_specs=pl.BlockSpec((BI, F), lambda i: (i, 0)),
        out_shape=jax.ShapeDtypeStruct((N, F), jnp.float32),
    )(nx, emb32, waa, baa)


def _matmul_body(x_ref, w_ref, deg_ref, h4_ref):
    dis = _dis_block(deg_ref)
    h = jnp.dot(x_ref[...], w_ref[...], preferred_element_type=jnp.float32)
    hp = h * dis
    for p in range(NPASS):
        h4_ref[p] = hp[:, p * FC:(p + 1) * FC]


def _matmul_scale(x, w, deg_part):
    return pl.pallas_call(
        _matmul_body,
        grid=(NBLK,),
        in_specs=[
            pl.BlockSpec((BI, F), lambda i: (i, 0)),
            pl.BlockSpec((F, F), lambda i: (0, 0)),
            pl.BlockSpec((2, BI, FC), lambda i: (0, i, 0)),
        ],
        out_specs=pl.BlockSpec((NPASS, BI, FC), lambda i: (0, i, 0)),
        out_shape=jax.ShapeDtypeStruct((NPASS, N, FC), jnp.float32),
    )(x, w, deg_part)


def _epilogue_body(agg_ref, h4_ref, deg_ref, b_ref, res_ref, out_ref,
                   *, has_res):
    dis = _dis_block(deg_ref)
    agg = jnp.concatenate([agg_ref[p] for p in range(NPASS)], axis=1)
    hp = jnp.concatenate([h4_ref[p] for p in range(NPASS)], axis=1)
    y = jnp.maximum(dis * (agg + hp) + b_ref[...], 0.0)
    if has_res:
        y = y + res_ref[...]
    out_ref[...] = y


def _epilogue(agg4, h4, deg_part, b, res, has_res):
    body = functools.partial(_epilogue_body, has_res=has_res)
    return pl.pallas_call(
        body,
        grid=(NBLK,),
        in_specs=[
            pl.BlockSpec((NPASS, BI, FC), lambda i: (0, i, 0)),
            pl.BlockSpec((NPASS, BI, FC), lambda i: (0, i, 0)),
            pl.BlockSpec((2, BI, FC), lambda i: (0, i, 0)),
            pl.BlockSpec((1, F), lambda i: (0, 0)),
            pl.BlockSpec((BI, F), lambda i: (i, 0)),
        ],
        out_specs=pl.BlockSpec((BI, F), lambda i: (i, 0)),
        out_shape=jax.ShapeDtypeStruct((N, F), jnp.float32),
    )(agg4, h4, deg_part, b, res)


def _readout_body(x_ref, batch_ref, wr1_ref, br1_ref, wr2_ref, br2_ref,
                  y_ref, g_ref):
    i = pl.program_id(0)

    @pl.when(i == 0)
    def _():
        g_ref[...] = jnp.full((G, F), -jnp.inf, jnp.float32)

    bb = batch_ref[...]
    xb = x_ref[...]
    for g in range(G):
        m = bb == g
        mg = jnp.max(jnp.where(m, xb, -jnp.inf), axis=0, keepdims=True)
        g_ref[pl.ds(g, 1), :] = jnp.maximum(g_ref[pl.ds(g, 1), :], mg)

    @pl.when(i == NBLK - 1)
    def _():
        gg = g_ref[...]
        h = jnp.maximum(
            jnp.dot(gg, wr1_ref[...], preferred_element_type=jnp.float32)
            + br1_ref[...], 0.0)
        y_ref[...] = jax.nn.sigmoid(
            jnp.dot(h, wr2_ref[...], preferred_element_type=jnp.float32)
            + br2_ref[...])


def _readout(x, batch, wr1, br1, wr2, br2):
    return pl.pallas_call(
        _readout_body,
        grid=(NBLK,),
        in_specs=[
            pl.BlockSpec((BI, F), lambda i: (i, 0)),
            pl.BlockSpec((BI, 1), lambda i: (i, 0)),
            pl.BlockSpec((F, 1024), lambda i: (0, 0)),
            pl.BlockSpec((1, 1024), lambda i: (0, 0)),
            pl.BlockSpec((1024, 256), lambda i: (0, 0)),
            pl.BlockSpec((1, 256), lambda i: (0, 0)),
        ],
        out_specs=[
            pl.BlockSpec((G, 256), lambda i: (0, 0)),
            pl.BlockSpec((G, F), lambda i: (0, 0)),
        ],
        out_shape=[
            jax.ShapeDtypeStruct((G, 256), jnp.float32),
            jax.ShapeDtypeStruct((G, F), jnp.float32),
        ],
    )(x, batch, wr1, br1, wr2, br2)


def kernel(native_x, edge_index, batch, emb, W_aa, b_aa, W_g1, b_g1,
           W_g2, b_g2, W_g3, b_g3, W_r1, b_r1, W_r2, b_r2):
    nx = native_x.astype(jnp.int32)
    ei = edge_index.astype(jnp.int32)
    src = jnp.concatenate([ei[0], jnp.zeros((EPAD,), jnp.int32)])
    dst = jnp.concatenate([ei[1], jnp.full((EPAD,), N, jnp.int32)])
    src2 = src.reshape(EC, 8, 128)
    dst2 = dst.reshape(EC, 8, 128)
    emb32 = jnp.pad(emb, ((0, 32 - emb.shape[0]), (0, 0)))

    deg_part = _sc_degree(dst2)
    x = _embed(nx.reshape(N, 1), emb32, W_aa, b_aa.reshape(1, F))
    for W, b, has_res in ((W_g1, b_g1, False), (W_g2, b_g2, True),
                          (W_g3, b_g3, True)):
        h4 = _matmul_scale(x, W, deg_part)
        agg4 = _sc_agg(h4, src2, dst2)
        x = _epilogue(agg4, h4, deg_part, b.reshape(1, F), x, has_res)
    y_pred, g_level_feat = _readout(x, batch.astype(jnp.int32).reshape(N, 1), W_r1,
                                    b_r1.reshape(1, 1024), W_r2,
                                    b_r2.reshape(1, 256))
    return (y_pred, g_level_feat)


# speedup vs baseline: 1.2941x; 1.0253x over previous
import functools

import jax
import jax.numpy as jnp
from jax import lax
from jax.experimental import pallas as pl
from jax.experimental.pallas import tpu as pltpu
from jax.experimental.pallas import tpu_sc as plsc

N = 10000
E = 160000
G = 16
F = 512
FC = 128
NPASS = F // FC
EC = 160
EPAD = EC * 1024 - E
N_ACC = 10240
STRIPE = N_ACC // 16
BI = 1000
NBLK = N // BI

_mesh = plsc.VectorSubcoreMesh(
    core_axis_name="c", subcore_axis_name="s", num_cores=2, num_subcores=16)


def _zero_vmem(ref, nrows, ncols):
    zv = jnp.zeros((16,), jnp.float32)

    def body(i, _):
        for j in range(ncols // 16):
            ref[i, pl.ds(j * 16, 16)] = zv
        return 0

    lax.fori_loop(0, nrows, body, 0)


@functools.partial(
    pl.kernel,
    out_type=jax.ShapeDtypeStruct((2, N_ACC, FC), jnp.float32),
    mesh=_mesh,
    scratch_types=[
        pltpu.VMEM((8, 128), jnp.int32),
        pltpu.VMEM((128, FC), jnp.float32),
        pltpu.VMEM((128, FC), jnp.float32),
        pltpu.VMEM_SHARED((N_ACC, FC), jnp.float32),
    ],
)
def _sc_degree(dst2h, deg_out, didx_v, ones_v, stage_v, acc_sh):
    c = lax.axis_index("c")
    s = lax.axis_index("s")

    _zero_vmem(stage_v, 128, FC)
    ov = jnp.ones((16,), jnp.float32)

    def fill_ones(i, _):
        for j in range(FC // 16):
            ones_v[i, pl.ds(j * 16, 16)] = ov
        return 0

    lax.fori_loop(0, 128, fill_ones, 0)

    for t in range(STRIPE // 128):
        pltpu.sync_copy(stage_v, acc_sh.at[pl.ds(s * STRIPE + t * 128, 128)])
    plsc.subcore_barrier()

    w = s * 2 + c

    def step(k, _):
        j = w + 32 * k
        pltpu.sync_copy(dst2h.at[j], didx_v)
        for r in range(8):
            pltpu.sync_copy(ones_v, acc_sh.at[didx_v.at[r]], add=True)
        return 0

    lax.fori_loop(0, EC // 32, step, 0)
    plsc.subcore_barrier()

    for t in range(STRIPE // 128):
        r0 = s * STRIPE + t * 128
        pltpu.sync_copy(acc_sh.at[pl.ds(r0, 128)], stage_v)
        pltpu.sync_copy(stage_v, deg_out.at[c, pl.ds(r0, 128)])


@functools.partial(
    pl.kernel,
    out_type=jax.ShapeDtypeStruct((NPASS, N_ACC, FC), jnp.float32),
    mesh=_mesh,
    scratch_types=[
        pltpu.VMEM((2, 8, 128), jnp.int32),
        pltpu.VMEM((2, 8, 128), jnp.int32),
        pltpu.VMEM((2, 128, FC), jnp.float32),
        pltpu.VMEM((64, FC), jnp.float32),
        pltpu.VMEM_SHARED((N_ACC, FC), jnp.float32),
        pltpu.SemaphoreType.DMA,
        pltpu.SemaphoreType.DMA,
    ],
)
def _sc_agg(h4, src2, dst2, agg4, sidx_v, didx_v, rows_v, stage_v,
            acc_sh, sem, sem2):
    c = lax.axis_index("c")
    s = lax.axis_index("s")
    _zero_vmem(stage_v, 64, FC)

    def run_pass(p):
        for t in range(STRIPE // 64):
            pltpu.sync_copy(stage_v,
                            acc_sh.at[pl.ds(s * STRIPE + t * 64, 64)])
        plsc.subcore_barrier()

        pltpu.async_copy(src2.at[s], sidx_v.at[0], sem2)
        pltpu.async_copy(dst2.at[s], didx_v.at[0], sem2)
        pltpu.async_copy(src2.at[s + 16], sidx_v.at[1], sem2)
        pltpu.async_copy(dst2.at[s + 16], didx_v.at[1], sem2)
        pltpu.make_async_copy(src2.at[s], sidx_v.at[0], sem2).wait()
        pltpu.make_async_copy(dst2.at[s], didx_v.at[0], sem2).wait()
        pltpu.async_copy(h4.at[p].at[sidx_v.at[0, 0]], rows_v.at[0], sem)

        def step(k, _):
            kp = lax.rem(k, 2)
            for r in range(8):
                if r < 7:
                    pltpu.async_copy(h4.at[p].at[sidx_v.at[kp, r + 1]],
                                     rows_v.at[(r + 1) % 2], sem)
                else:
                    @pl.when(k + 1 < EC // 16)
                    def _():
                        kq = 1 - kp
                        j1 = s + 16 * (k + 1)
                        pltpu.make_async_copy(src2.at[j1], sidx_v.at[kq],
                                              sem2).wait()
                        pltpu.make_async_copy(dst2.at[j1], didx_v.at[kq],
                                              sem2).wait()
                        pltpu.async_copy(h4.at[p].at[sidx_v.at[kq, 0]],
                                         rows_v.at[0], sem)
                pltpu.make_async_copy(h4.at[p].at[sidx_v.at[kp, r]],
                                      rows_v.at[r % 2], sem).wait()
                pltpu.sync_copy(rows_v.at[r % 2],
                                acc_sh.at[didx_v.at[kp, r]], add=True)

            @pl.when(k + 2 < EC // 16)
            def _():
                j2 = s + 16 * (k + 2)
                pltpu.async_copy(src2.at[j2], sidx_v.at[kp], sem2)
                pltpu.async_copy(dst2.at[j2], didx_v.at[kp], sem2)
            return 0

        lax.fori_loop(0, EC // 16, step, 0)
        plsc.subcore_barrier()

        for t in range(STRIPE // 64):
            r0 = s * STRIPE + t * 64
            pltpu.sync_copy(acc_sh.at[pl.ds(r0, 64)], stage_v)
            pltpu.sync_copy(stage_v, agg4.at[p, pl.ds(r0, 64)])
        plsc.subcore_barrier()
        _zero_vmem(stage_v, 64, FC)
        plsc.subcore_barrier()

    for p_local in range(2):
        for cval in range(2):
            @pl.when(c == cval)
            def _(p=2 * cval + p_local):
                run_pass(p)


def _dis_block(deg_ref):
    deg = deg_ref[0, :, 0:1] + deg_ref[1, :, 0:1] + 1.0
    return lax.rsqrt(deg)


def _embed_body(nx_ref, emb_ref, waa_ref, baa_ref, out_ref):
    nx = nx_ref[...]
    oh = (nx == lax.broadcasted_iota(jnp.int32, (BI, 32), 1)
          ).astype(jnp.float32)
    table = jnp.dot(emb_ref[...], waa_ref[...],
                    preferred_element_type=jnp.float32)
    x0 = jnp.dot(oh, table, preferred_element_type=jnp.float32) + baa_ref[...]
    out_ref[...] = jnp.maximum(x0, 0.0)


def _embed(nx, emb32, waa, baa):
    return pl.pallas_call(
        _embed_body,
        grid=(NBLK,),
        in_specs=[
            pl.BlockSpec((BI, 1), lambda i: (i, 0)),
            pl.BlockSpec((32, 96), lambda i: (0, 0)),
            pl.BlockSpec((96, F), lambda i: (0, 0)),
            pl.BlockSpec((1, F), lambda i: (0, 0)),
        ],
        out_specs=pl.BlockSpec((BI, F), lambda i: (i, 0)),
        out_shape=jax.ShapeDtypeStruct((N, F), jnp.float32),
    )(nx, emb32, waa, baa)


def _matmul_body(x_ref, w_ref, deg_ref, h4_ref):
    dis = _dis_block(deg_ref)
    h = jnp.dot(x_ref[...], w_ref[...], preferred_element_type=jnp.float32)
    hp = h * dis
    for p in range(NPASS):
        h4_ref[p] = hp[:, p * FC:(p + 1) * FC]


def _matmul_scale(x, w, deg_part):
    return pl.pallas_call(
        _matmul_body,
        grid=(NBLK,),
        in_specs=[
            pl.BlockSpec((BI, F), lambda i: (i, 0)),
            pl.BlockSpec((F, F), lambda i: (0, 0)),
            pl.BlockSpec((2, BI, FC), lambda i: (0, i, 0)),
        ],
        out_specs=pl.BlockSpec((NPASS, BI, FC), lambda i: (0, i, 0)),
        out_shape=jax.ShapeDtypeStruct((NPASS, N, FC), jnp.float32),
    )(x, w, deg_part)


def _epilogue_body(agg_ref, h4_ref, deg_ref, b_ref, res_ref, out_ref,
                   *, has_res):
    dis = _dis_block(deg_ref)
    agg = jnp.concatenate([agg_ref[p] for p in range(NPASS)], axis=1)
    hp = jnp.concatenate([h4_ref[p] for p in range(NPASS)], axis=1)
    y = jnp.maximum(dis * (agg + hp) + b_ref[...], 0.0)
    if has_res:
        y = y + res_ref[...]
    out_ref[...] = y


def _epilogue(agg4, h4, deg_part, b, res, has_res):
    body = functools.partial(_epilogue_body, has_res=has_res)
    return pl.pallas_call(
        body,
        grid=(NBLK,),
        in_specs=[
            pl.BlockSpec((NPASS, BI, FC), lambda i: (0, i, 0)),
            pl.BlockSpec((NPASS, BI, FC), lambda i: (0, i, 0)),
            pl.BlockSpec((2, BI, FC), lambda i: (0, i, 0)),
            pl.BlockSpec((1, F), lambda i: (0, 0)),
            pl.BlockSpec((BI, F), lambda i: (i, 0)),
        ],
        out_specs=pl.BlockSpec((BI, F), lambda i: (i, 0)),
        out_shape=jax.ShapeDtypeStruct((N, F), jnp.float32),
    )(agg4, h4, deg_part, b, res)


def _readout_body(x_ref, batch_ref, wr1_ref, br1_ref, wr2_ref, br2_ref,
                  y_ref, g_ref):
    i = pl.program_id(0)

    @pl.when(i == 0)
    def _():
        g_ref[...] = jnp.full((G, F), -jnp.inf, jnp.float32)

    bb = batch_ref[...]
    xb = x_ref[...]
    for g in range(G):
        m = bb == g
        mg = jnp.max(jnp.where(m, xb, -jnp.inf), axis=0, keepdims=True)
        g_ref[pl.ds(g, 1), :] = jnp.maximum(g_ref[pl.ds(g, 1), :], mg)

    @pl.when(i == NBLK - 1)
    def _():
        gg = g_ref[...]
        h = jnp.maximum(
            jnp.dot(gg, wr1_ref[...], preferred_element_type=jnp.float32)
            + br1_ref[...], 0.0)
        y_ref[...] = jax.nn.sigmoid(
            jnp.dot(h, wr2_ref[...], preferred_element_type=jnp.float32)
            + br2_ref[...])


def _readout(x, batch, wr1, br1, wr2, br2):
    return pl.pallas_call(
        _readout_body,
        grid=(NBLK,),
        in_specs=[
            pl.BlockSpec((BI, F), lambda i: (i, 0)),
            pl.BlockSpec((BI, 1), lambda i: (i, 0)),
            pl.BlockSpec((F, 1024), lambda i: (0, 0)),
            pl.BlockSpec((1, 1024), lambda i: (0, 0)),
            pl.BlockSpec((1024, 256), lambda i: (0, 0)),
            pl.BlockSpec((1, 256), lambda i: (0, 0)),
        ],
        out_specs=[
            pl.BlockSpec((G, 256), lambda i: (0, 0)),
            pl.BlockSpec((G, F), lambda i: (0, 0)),
        ],
        out_shape=[
            jax.ShapeDtypeStruct((G, 256), jnp.float32),
            jax.ShapeDtypeStruct((G, F), jnp.float32),
        ],
    )(x, batch, wr1, br1, wr2, br2)


def kernel(native_x, edge_index, batch, emb, W_aa, b_aa, W_g1, b_g1,
           W_g2, b_g2, W_g3, b_g3, W_r1, b_r1, W_r2, b_r2):
    nx = native_x.astype(jnp.int32)
    ei = edge_index.astype(jnp.int32)
    src = jnp.concatenate([ei[0], jnp.zeros((EPAD,), jnp.int32)])
    dst = jnp.concatenate([ei[1], jnp.full((EPAD,), N, jnp.int32)])
    src2 = src.reshape(EC, 8, 128)
    dst2 = dst.reshape(EC, 8, 128)
    emb32 = jnp.pad(emb, ((0, 32 - emb.shape[0]), (0, 0)))

    deg_part = _sc_degree(dst2)
    x = _embed(nx.reshape(N, 1), emb32, W_aa, b_aa.reshape(1, F))
    for W, b, has_res in ((W_g1, b_g1, False), (W_g2, b_g2, True),
                          (W_g3, b_g3, True)):
        h4 = _matmul_scale(x, W, deg_part)
        agg4 = _sc_agg(h4, src2, dst2)
        x = _epilogue(agg4, h4, deg_part, b.reshape(1, F), x, has_res)
    y_pred, g_level_feat = _readout(x, batch.astype(jnp.int32).reshape(N, 1), W_r1,
                                    b_r1.reshape(1, 1024), W_r2,
                                    b_r2.reshape(1, 256))
    return (y_pred, g_level_feat)


# speedup vs baseline: 1.2992x; 1.0040x over previous
import functools

import jax
import jax.numpy as jnp
from jax import lax
from jax.experimental import pallas as pl
from jax.experimental.pallas import tpu as pltpu
from jax.experimental.pallas import tpu_sc as plsc

N = 10000
E = 160000
G = 16
F = 512
FC = 128
NPASS = F // FC
EC = 160
EPAD = EC * 1024 - E
N_ACC = 10240
STRIPE = N_ACC // 16
BI = 1000
NBLK = N // BI

_mesh = plsc.VectorSubcoreMesh(
    core_axis_name="c", subcore_axis_name="s", num_cores=2, num_subcores=16)


def _zero_vmem(ref, nrows, ncols):
    zv = jnp.zeros((16,), jnp.float32)

    def body(i, _):
        for j in range(ncols // 16):
            ref[i, pl.ds(j * 16, 16)] = zv
        return 0

    lax.fori_loop(0, nrows, body, 0)


@functools.partial(
    pl.kernel,
    out_type=jax.ShapeDtypeStruct((2, N_ACC, FC), jnp.float32),
    mesh=_mesh,
    scratch_types=[
        pltpu.VMEM((8, 128), jnp.int32),
        pltpu.VMEM((128, FC), jnp.float32),
        pltpu.VMEM((128, FC), jnp.float32),
        pltpu.VMEM_SHARED((N_ACC, FC), jnp.float32),
    ],
)
def _sc_degree(dst2h, deg_out, didx_v, ones_v, stage_v, acc_sh):
    c = lax.axis_index("c")
    s = lax.axis_index("s")

    _zero_vmem(stage_v, 128, FC)
    ov = jnp.ones((16,), jnp.float32)

    def fill_ones(i, _):
        for j in range(FC // 16):
            ones_v[i, pl.ds(j * 16, 16)] = ov
        return 0

    lax.fori_loop(0, 128, fill_ones, 0)

    for t in range(STRIPE // 128):
        pltpu.sync_copy(stage_v, acc_sh.at[pl.ds(s * STRIPE + t * 128, 128)])
    plsc.subcore_barrier()

    w = s * 2 + c

    def step(k, _):
        j = w + 32 * k
        pltpu.sync_copy(dst2h.at[j], didx_v)
        for r in range(8):
            pltpu.sync_copy(ones_v, acc_sh.at[didx_v.at[r]], add=True)
        return 0

    lax.fori_loop(0, EC // 32, step, 0)
    plsc.subcore_barrier()

    r0 = s * STRIPE
    pltpu.sync_copy(acc_sh.at[pl.ds(r0, STRIPE)],
                    deg_out.at[c, pl.ds(r0, STRIPE)])


@functools.partial(
    pl.kernel,
    out_type=jax.ShapeDtypeStruct((NPASS, N_ACC, FC), jnp.float32),
    mesh=_mesh,
    scratch_types=[
        pltpu.VMEM((2, 8, 128), jnp.int32),
        pltpu.VMEM((2, 8, 128), jnp.int32),
        pltpu.VMEM((2, 128, FC), jnp.float32),
        pltpu.VMEM((64, FC), jnp.float32),
        pltpu.VMEM_SHARED((N_ACC, FC), jnp.float32),
        pltpu.SemaphoreType.DMA,
        pltpu.SemaphoreType.DMA,
    ],
)
def _sc_agg(h4, src2, dst2, agg4, sidx_v, didx_v, rows_v, stage_v,
            acc_sh, sem, sem2):
    c = lax.axis_index("c")
    s = lax.axis_index("s")
    _zero_vmem(stage_v, 64, FC)

    def run_pass(p):
        for t in range(STRIPE // 64):
            pltpu.sync_copy(stage_v,
                            acc_sh.at[pl.ds(s * STRIPE + t * 64, 64)])
        plsc.subcore_barrier()

        pltpu.async_copy(src2.at[s], sidx_v.at[0], sem2)
        pltpu.async_copy(dst2.at[s], didx_v.at[0], sem2)
        pltpu.async_copy(src2.at[s + 16], sidx_v.at[1], sem2)
        pltpu.async_copy(dst2.at[s + 16], didx_v.at[1], sem2)
        pltpu.make_async_copy(src2.at[s], sidx_v.at[0], sem2).wait()
        pltpu.make_async_copy(dst2.at[s], didx_v.at[0], sem2).wait()
        pltpu.async_copy(h4.at[p].at[sidx_v.at[0, 0]], rows_v.at[0], sem)

        def step(k, _):
            kp = lax.rem(k, 2)
            for r in range(8):
                if r < 7:
                    pltpu.async_copy(h4.at[p].at[sidx_v.at[kp, r + 1]],
                                     rows_v.at[(r + 1) % 2], sem)
                else:
                    @pl.when(k + 1 < EC // 16)
                    def _():
                        kq = 1 - kp
                        j1 = s + 16 * (k + 1)
                        pltpu.make_async_copy(src2.at[j1], sidx_v.at[kq],
                                              sem2).wait()
                        pltpu.make_async_copy(dst2.at[j1], didx_v.at[kq],
                                              sem2).wait()
                        pltpu.async_copy(h4.at[p].at[sidx_v.at[kq, 0]],
                                         rows_v.at[0], sem)
                pltpu.make_async_copy(h4.at[p].at[sidx_v.at[kp, r]],
                                      rows_v.at[r % 2], sem).wait()
                pltpu.sync_copy(rows_v.at[r % 2],
                                acc_sh.at[didx_v.at[kp, r]], add=True)

            @pl.when(k + 2 < EC // 16)
            def _():
                j2 = s + 16 * (k + 2)
                pltpu.async_copy(src2.at[j2], sidx_v.at[kp], sem2)
                pltpu.async_copy(dst2.at[j2], didx_v.at[kp], sem2)
            return 0

        lax.fori_loop(0, EC // 16, step, 0)
        plsc.subcore_barrier()

        r0 = s * STRIPE
        pltpu.sync_copy(acc_sh.at[pl.ds(r0, STRIPE)],
                        agg4.at[p, pl.ds(r0, STRIPE)])
        plsc.subcore_barrier()
        _zero_vmem(stage_v, 64, FC)
        plsc.subcore_barrier()

    for p_local in range(2):
        for cval in range(2):
            @pl.when(c == cval)
            def _(p=2 * cval + p_local):
                run_pass(p)


def _dis_block(deg_ref):
    deg = deg_ref[0, :, 0:1] + deg_ref[1, :, 0:1] + 1.0
    return lax.rsqrt(deg)


def _embed_body(nx_ref, emb_ref, waa_ref, baa_ref, out_ref):
    nx = nx_ref[...]
    oh = (nx == lax.broadcasted_iota(jnp.int32, (BI, 32), 1)
          ).astype(jnp.float32)
    table = jnp.dot(emb_ref[...], waa_ref[...],
                    preferred_element_type=jnp.float32)
    x0 = jnp.dot(oh, table, preferred_element_type=jnp.float32) + baa_ref[...]
    out_ref[...] = jnp.maximum(x0, 0.0)


def _embed(nx, emb32, waa, baa):
    return pl.pallas_call(
        _embed_body,
        grid=(NBLK,),
        in_specs=[
            pl.BlockSpec((BI, 1), lambda i: (i, 0)),
            pl.BlockSpec((32, 96), lambda i: (0, 0)),
            pl.BlockSpec((96, F), lambda i: (0, 0)),
            pl.BlockSpec((1, F), lambda i: (0, 0)),
        ],
        out_specs=pl.BlockSpec((BI, F), lambda i: (i, 0)),
        out_shape=jax.ShapeDtypeStruct((N, F), jnp.float32),
    )(nx, emb32, waa, baa)


def _matmul_body(x_ref, w_ref, deg_ref, h4_ref):
    dis = _dis_block(deg_ref)
    h = jnp.dot(x_ref[...], w_ref[...], preferred_element_type=jnp.float32)
    hp = h * dis
    for p in range(NPASS):
        h4_ref[p] = hp[:, p * FC:(p + 1) * FC]


def _matmul_scale(x, w, deg_part):
    return pl.pallas_call(
        _matmul_body,
        grid=(NBLK,),
        in_specs=[
            pl.BlockSpec((BI, F), lambda i: (i, 0)),
            pl.BlockSpec((F, F), lambda i: (0, 0)),
            pl.BlockSpec((2, BI, FC), lambda i: (0, i, 0)),
        ],
        out_specs=pl.BlockSpec((NPASS, BI, FC), lambda i: (0, i, 0)),
        out_shape=jax.ShapeDtypeStruct((NPASS, N, FC), jnp.float32),
    )(x, w, deg_part)


def _epilogue_body(agg_ref, h4_ref, deg_ref, b_ref, res_ref, out_ref,
                   *, has_res):
    dis = _dis_block(deg_ref)
    agg = jnp.concatenate([agg_ref[p] for p in range(NPASS)], axis=1)
    hp = jnp.concatenate([h4_ref[p] for p in range(NPASS)], axis=1)
    y = jnp.maximum(dis * (agg + hp) + b_ref[...], 0.0)
    if has_res:
        y = y + res_ref[...]
    out_ref[...] = y


def _epilogue(agg4, h4, deg_part, b, res, has_res):
    body = functools.partial(_epilogue_body, has_res=has_res)
    return pl.pallas_call(
        body,
        grid=(NBLK,),
        in_specs=[
            pl.BlockSpec((NPASS, BI, FC), lambda i: (0, i, 0)),
            pl.BlockSpec((NPASS, BI, FC), lambda i: (0, i, 0)),
            pl.BlockSpec((2, BI, FC), lambda i: (0, i, 0)),
            pl.BlockSpec((1, F), lambda i: (0, 0)),
            pl.BlockSpec((BI, F), lambda i: (i, 0)),
        ],
        out_specs=pl.BlockSpec((BI, F), lambda i: (i, 0)),
        out_shape=jax.ShapeDtypeStruct((N, F), jnp.float32),
    )(agg4, h4, deg_part, b, res)


def _readout_body(x_ref, batch_ref, wr1_ref, br1_ref, wr2_ref, br2_ref,
                  y_ref, g_ref):
    i = pl.program_id(0)

    @pl.when(i == 0)
    def _():
        g_ref[...] = jnp.full((G, F), -jnp.inf, jnp.float32)

    bb = batch_ref[...]
    xb = x_ref[...]
    for g in range(G):
        m = bb == g
        mg = jnp.max(jnp.where(m, xb, -jnp.inf), axis=0, keepdims=True)
        g_ref[pl.ds(g, 1), :] = jnp.maximum(g_ref[pl.ds(g, 1), :], mg)

    @pl.when(i == NBLK - 1)
    def _():
        gg = g_ref[...]
        h = jnp.maximum(
            jnp.dot(gg, wr1_ref[...], preferred_element_type=jnp.float32)
            + br1_ref[...], 0.0)
        y_ref[...] = jax.nn.sigmoid(
            jnp.dot(h, wr2_ref[...], preferred_element_type=jnp.float32)
            + br2_ref[...])


def _readout(x, batch, wr1, br1, wr2, br2):
    return pl.pallas_call(
        _readout_body,
        grid=(NBLK,),
        in_specs=[
            pl.BlockSpec((BI, F), lambda i: (i, 0)),
            pl.BlockSpec((BI, 1), lambda i: (i, 0)),
            pl.BlockSpec((F, 1024), lambda i: (0, 0)),
            pl.BlockSpec((1, 1024), lambda i: (0, 0)),
            pl.BlockSpec((1024, 256), lambda i: (0, 0)),
            pl.BlockSpec((1, 256), lambda i: (0, 0)),
        ],
        out_specs=[
            pl.BlockSpec((G, 256), lambda i: (0, 0)),
            pl.BlockSpec((G, F), lambda i: (0, 0)),
        ],
        out_shape=[
            jax.ShapeDtypeStruct((G, 256), jnp.float32),
            jax.ShapeDtypeStruct((G, F), jnp.float32),
        ],
    )(x, batch, wr1, br1, wr2, br2)


def kernel(native_x, edge_index, batch, emb, W_aa, b_aa, W_g1, b_g1,
           W_g2, b_g2, W_g3, b_g3, W_r1, b_r1, W_r2, b_r2):
    nx = native_x.astype(jnp.int32)
    ei = edge_index.astype(jnp.int32)
    src = jnp.concatenate([ei[0], jnp.zeros((EPAD,), jnp.int32)])
    dst = jnp.concatenate([ei[1], jnp.full((EPAD,), N, jnp.int32)])
    src2 = src.reshape(EC, 8, 128)
    dst2 = dst.reshape(EC, 8, 128)
    emb32 = jnp.pad(emb, ((0, 32 - emb.shape[0]), (0, 0)))

    deg_part = _sc_degree(dst2)
    x = _embed(nx.reshape(N, 1), emb32, W_aa, b_aa.reshape(1, F))
    for W, b, has_res in ((W_g1, b_g1, False), (W_g2, b_g2, True),
                          (W_g3, b_g3, True)):
        h4 = _matmul_scale(x, W, deg_part)
        agg4 = _sc_agg(h4, src2, dst2)
        x = _epilogue(agg4, h4, deg_part, b.reshape(1, F), x, has_res)
    y_pred, g_level_feat = _readout(x, batch.astype(jnp.int32).reshape(N, 1), W_r1,
                                    b_r1.reshape(1, 1024), W_r2,
                                    b_r2.reshape(1, 256))
    return (y_pred, g_level_feat)
